# baseline (speedup 1.0000x reference)
import functools

import jax
import jax.numpy as jnp
from jax import lax
from jax.experimental import pallas as pl
from jax.experimental.pallas import tpu as pltpu
from jax.experimental.pallas import tpu_sc as plsc

_N = 10000
_E = 160000
_D = 256
_W = 64
_NG = 4
_NT = 16
_NC = 2
_CH = 128
_EPT = _E // _NT
_NCH = 79
_PAD = _NCH * _CH - _EPT
_NA = _N + 8
_RPT = 632
_RZLAST = _NA - 15 * _RPT
_RWLAST = _N - 15 * _RPT
_RB = 1000
_G = _N // _RB


def _agg_body(x_hbm, src_hbm, dst_hbm, z_hbm, out_hbm,
              sidx, didx, s128a, d128a, s128b, d128b, rows_a, rows_b,
              acc, gsem_a, gsem_b):
    c = lax.axis_index("c")
    s = lax.axis_index("s")
    pltpu.sync_copy(src_hbm.at[s], sidx)
    pltpu.sync_copy(dst_hbm.at[s], didx)

    for ph in range(2):
        g = 2 * c + ph

        @pl.when(s < _NT - 1)
        def _():
            pltpu.sync_copy(z_hbm, acc.at[pl.ds(s * _RPT, _RPT)])

        @pl.when(s == _NT - 1)
        def _():
            pltpu.sync_copy(z_hbm.at[pl.ds(0, _RZLAST)],
                            acc.at[pl.ds(15 * _RPT, _RZLAST)])

        plsc.subcore_barrier()

        goff = g * _N

        def cidx(j, s128, d128):
            for k in range(_CH // 16):
                s128[pl.ds(k * 16, 16)] = sidx[j, pl.ds(k * 16, 16)] + goff
                d128[pl.ds(k * 16, 16)] = didx[j, pl.ds(k * 16, 16)]

        cidx(0, s128a, d128a)
        pltpu.async_copy(x_hbm.at[s128a], rows_a, gsem_a)

        def pair(k, carry):
            j = 2 * k
            cidx(j + 1, s128b, d128b)
            pltpu.async_copy(x_hbm.at[s128b], rows_b, gsem_b)
            pltpu.make_async_copy(x_hbm.at[s128a], rows_a, gsem_a).wait()
            pltpu.sync_copy(rows_a, acc.at[d128a], add=True)
            cidx(j + 2, s128a, d128a)
            pltpu.async_copy(x_hbm.at[s128a], rows_a, gsem_a)
            pltpu.make_async_copy(x_hbm.at[s128b], rows_b, gsem_b).wait()
            pltpu.sync_copy(rows_b, acc.at[d128b], add=True)
            return carry

        lax.fori_loop(0, (_NCH - 1) // 2, pair, 0)
        pltpu.make_async_copy(x_hbm.at[s128a], rows_a, gsem_a).wait()
        pltpu.sync_copy(rows_a, acc.at[d128a], add=True)
        plsc.subcore_barrier()

        @pl.when(s < _NT - 1)
        def _():
            pltpu.sync_copy(acc.at[pl.ds(s * _RPT, _RPT)],
                            out_hbm.at[pl.ds(goff + s * _RPT, _RPT)])

        @pl.when(s == _NT - 1)
        def _():
            pltpu.sync_copy(acc.at[pl.ds(15 * _RPT, _RWLAST)],
                            out_hbm.at[pl.ds(goff + 15 * _RPT, _RWLAST)])


@functools.cache
def _make_agg(interpret: bool = False):
    return pl.kernel(
        _agg_body,
        out_type=jax.ShapeDtypeStruct((_NG * _N, _W), jnp.float32),
        mesh=plsc.VectorSubcoreMesh(core_axis_name="c", subcore_axis_name="s"),
        scratch_types=[
            pltpu.VMEM((_NCH, _CH), jnp.int32),
            pltpu.VMEM((_NCH, _CH), jnp.int32),
            pltpu.VMEM((_CH,), jnp.int32),
            pltpu.VMEM((_CH,), jnp.int32),
            pltpu.VMEM((_CH,), jnp.int32),
            pltpu.VMEM((_CH,), jnp.int32),
            pltpu.VMEM((_CH, _W), jnp.float32),
            pltpu.VMEM((_CH, _W), jnp.float32),
            pltpu.VMEM_SHARED((_NA, _W), jnp.float32),
            pltpu.SemaphoreType.DMA,
            pltpu.SemaphoreType.DMA,
        ],
        compiler_params=pltpu.CompilerParams(use_tc_tiling_on_sc=False),
        interpret=interpret,
    )


def _cat(ref):
    return jnp.concatenate([ref[g] for g in range(_NG)], axis=1)


def _split(ref, h):
    for g in range(_NG):
        ref[g] = h[:, g * _W:(g + 1) * _W]


def _s1_body(s_ref, x_ref, c_ref, wl_ref, wr_ref, b_ref, o_ref):
    ic = 1.0 / jnp.maximum(c_ref[...], 1.0)
    S = _cat(s_ref)
    X = _cat(x_ref)
    h = (jnp.dot(S * ic, wl_ref[...], preferred_element_type=jnp.float32)
         + jnp.dot(X, wr_ref[...], preferred_element_type=jnp.float32)
         + b_ref[...])
    _split(o_ref, jnp.maximum(h, 0.0))


def _s2_body(s_ref, x_ref, c_ref, wl_ref, wr_ref, b_ref, u_ref, h_ref):
    cnt = c_ref[...]
    ic = 1.0 / jnp.maximum(cnt, 1.0)
    dinv = lax.rsqrt(cnt + 1.0)
    S = _cat(s_ref)
    X = _cat(x_ref)
    h = (jnp.dot(S * ic, wl_ref[...], preferred_element_type=jnp.float32)
         + jnp.dot(X, wr_ref[...], preferred_element_type=jnp.float32)
         + b_ref[...])
    _split(u_ref, dinv * h)
    _split(h_ref, h)


def _s3_body(g_ref, x_ref, c_ref, ws_ref, b_ref, v_ref, o_ref):
    dinv = lax.rsqrt(c_ref[...] + 1.0)
    Gm = _cat(g_ref)
    X = _cat(x_ref)
    P = dinv * Gm + (dinv * dinv) * X
    xs = jnp.maximum(
        jnp.dot(P, ws_ref[...], preferred_element_type=jnp.float32) + b_ref[...],
        0.0)
    _split(v_ref, dinv * xs)
    _split(o_ref, xs)


def _s4_body(g_ref, x_ref, c_ref, wm_ref, bm_ref, wv_ref, bv_ref, mu_ref, lv_ref):
    dinv = lax.rsqrt(c_ref[...] + 1.0)
    Gm = _cat(g_ref)
    X = _cat(x_ref)
    t = dinv * Gm + (dinv * dinv) * X
    mu_ref[...] = jnp.dot(t, wm_ref[...], preferred_element_type=jnp.float32) + bm_ref[...]
    lv_ref[...] = jnp.dot(t, wv_ref[...], preferred_element_type=jnp.float32) + bv_ref[...]


_FEAT = pl.BlockSpec((_NG, _RB, _W), lambda i: (0, i, 0))
_COL = pl.BlockSpec((_RB, 1), lambda i: (i, 0))
_WSPEC = pl.BlockSpec((_D, _D), lambda i: (0, 0))
_B = pl.BlockSpec((1, _D), lambda i: (0, 0))
_FULL = pl.BlockSpec((_RB, _D), lambda i: (i, 0))
_FSHAPE = jax.ShapeDtypeStruct((_NG, _N, _W), jnp.float32)
_OSHAPE = jax.ShapeDtypeStruct((_N, _D), jnp.float32)


@functools.cache
def _make_stages(interpret: bool = False):
    s1 = pl.pallas_call(
        _s1_body, grid=(_G,),
        in_specs=[_FEAT, _FEAT, _COL, _WSPEC, _WSPEC, _B],
        out_specs=_FEAT, out_shape=_FSHAPE, interpret=interpret)
    s2 = pl.pallas_call(
        _s2_body, grid=(_G,),
        in_specs=[_FEAT, _FEAT, _COL, _WSPEC, _WSPEC, _B],
        out_specs=(_FEAT, _FEAT), out_shape=(_FSHAPE, _FSHAPE),
        interpret=interpret)
    s3 = pl.pallas_call(
        _s3_body, grid=(_G,),
        in_specs=[_FEAT, _FEAT, _COL, _WSPEC, _B],
        out_specs=(_FEAT, _FEAT), out_shape=(_FSHAPE, _FSHAPE),
        interpret=interpret)
    s4 = pl.pallas_call(
        _s4_body, grid=(_G,),
        in_specs=[_FEAT, _FEAT, _COL, _WSPEC, _B, _WSPEC, _B],
        out_specs=(_FULL, _FULL), out_shape=(_OSHAPE, _OSHAPE),
        interpret=interpret)
    return s1, s2, s3, s4


def _run(x, edge_index, Wl1, bl1, Wr1, Wl2, bl2, Wr2, Ws, bs, Wm, bm, Wv, bv,
         interpret: bool = False):
    agg = _make_agg(interpret)
    s1, s2, s3, s4 = _make_stages(interpret)

    src = edge_index[0]
    dst = edge_index[1]
    srcp = jnp.concatenate(
        [src.reshape(_NT, _EPT),
         jnp.zeros((_NT, _PAD), jnp.int32)], axis=1).reshape(_NT, _NCH, _CH)
    dstp = jnp.concatenate(
        [dst.reshape(_NT, _EPT),
         jnp.full((_NT, _PAD), _N, jnp.int32)], axis=1).reshape(_NT, _NCH, _CH)
    zblk = jnp.zeros((_RPT, _W), jnp.float32)
    ones = jnp.ones((_NG * _N, _W), jnp.float32)

    X4 = jnp.moveaxis(x.reshape(_N, _NG, _W), 1, 0)

    cnt = agg(ones, srcp, dstp, zblk)[:_N, :1]
    S1 = agg(X4.reshape(_NG * _N, _W), srcp, dstp, zblk).reshape(_NG, _N, _W)
    H1 = s1(S1, X4, cnt, Wl1.T, Wr1.T, bl1.reshape(1, _D))
    S2 = agg(H1.reshape(_NG * _N, _W), srcp, dstp, zblk).reshape(_NG, _N, _W)
    U, H2 = s2(S2, H1, cnt, Wl2.T, Wr2.T, bl2.reshape(1, _D))
    Gm = agg(U.reshape(_NG * _N, _W), srcp, dstp, zblk).reshape(_NG, _N, _W)
    V, XS = s3(Gm, H2, cnt, Ws.T, bs.reshape(1, _D))
    Hh = agg(V.reshape(_NG * _N, _W), srcp, dstp, zblk).reshape(_NG, _N, _W)
    mu, lv = s4(Hh, XS, cnt, Wm.T, bm.reshape(1, _D), Wv.T, bv.reshape(1, _D))
    return (mu, lv)


def kernel(edge_emb_eq1, edge_index, Wl1, bl1, Wr1, Wl2, bl2, Wr2, Ws, bs,
           Wm, bm, Wv, bv):
    return _run(edge_emb_eq1, edge_index, Wl1, bl1, Wr1, Wl2, bl2, Wr2,
                Ws, bs, Wm, bm, Wv, bv)


# speedup vs baseline: 1.0944x; 1.0944x over previous
import functools

import jax
import jax.numpy as jnp
from jax import lax
from jax.experimental import pallas as pl
from jax.experimental.pallas import tpu as pltpu
from jax.experimental.pallas import tpu_sc as plsc

_N = 10000
_E = 160000
_D = 256
_W = 64
_NG = 4
_NT = 16
_NC = 2
_CH = 128
_EPT = _E // _NT
_NCH = 79
_PAD = _NCH * _CH - _EPT
_NA = _N + 8
_RPT = 632
_RZLAST = _NA - 15 * _RPT
_RWLAST = _N - 15 * _RPT
_RB = 1000
_G = _N // _RB


def _agg_body(x_hbm, src_hbm, dst_hbm, z_hbm, out_hbm,
              sidx, didx, s128a, d128a, s128b, d128b, s128c, d128c,
              rows_a, rows_b, rows_c, acc,
              gsem_a, gsem_b, gsem_c, ssem_a, ssem_b, ssem_c):
    c = lax.axis_index("c")
    s = lax.axis_index("s")
    pltpu.sync_copy(src_hbm.at[s], sidx)
    pltpu.sync_copy(dst_hbm.at[s], didx)

    bufs = ((s128a, d128a, rows_a, gsem_a, ssem_a),
            (s128b, d128b, rows_b, gsem_b, ssem_b),
            (s128c, d128c, rows_c, gsem_c, ssem_c))

    for ph in range(2):
        g = 2 * c + ph

        @pl.when(s < _NT - 1)
        def _():
            pltpu.sync_copy(z_hbm, acc.at[pl.ds(s * _RPT, _RPT)])

        @pl.when(s == _NT - 1)
        def _():
            pltpu.sync_copy(z_hbm.at[pl.ds(0, _RZLAST)],
                            acc.at[pl.ds(15 * _RPT, _RZLAST)])

        plsc.subcore_barrier()

        goff = g * _N

        def cidx(j, b):
            s128, d128 = b[0], b[1]
            for k in range(_CH // 16):
                s128[pl.ds(k * 16, 16)] = sidx[j, pl.ds(k * 16, 16)] + goff
                d128[pl.ds(k * 16, 16)] = didx[j, pl.ds(k * 16, 16)]

        def gat(j, b):
            cidx(j, b)
            pltpu.async_copy(x_hbm.at[b[0]], b[2], b[3])

        def wait_g(b):
            pltpu.make_async_copy(x_hbm.at[b[0]], b[2], b[3]).wait()

        def scat(b):
            pltpu.async_copy(b[2], acc.at[b[1]], b[4], add=True)

        def wait_s(b):
            pltpu.make_async_copy(b[2], acc.at[b[1]], b[4]).wait()

        gat(0, bufs[0])
        gat(1, bufs[1])
        wait_g(bufs[0])
        scat(bufs[0])
        gat(2, bufs[2])
        wait_g(bufs[1])
        scat(bufs[1])
        wait_s(bufs[0])
        gat(3, bufs[0])

        def tri(k, carry):
            j = 3 * k
            for t, (bj, bp) in enumerate(((2, 1), (0, 2), (1, 0))):
                jj = j + 2 + t
                wait_g(bufs[bj])
                scat(bufs[bj])
                wait_s(bufs[bp])
                gat(jj + 2, bufs[bp])
            return carry

        lax.fori_loop(0, (_NCH - 4) // 3, tri, 0)
        wait_g(bufs[2])
        scat(bufs[2])
        wait_g(bufs[0])
        scat(bufs[0])
        wait_s(bufs[1])
        wait_s(bufs[2])
        wait_s(bufs[0])
        plsc.subcore_barrier()

        @pl.when(s < _NT - 1)
        def _():
            pltpu.sync_copy(acc.at[pl.ds(s * _RPT, _RPT)],
                            out_hbm.at[pl.ds(goff + s * _RPT, _RPT)])

        @pl.when(s == _NT - 1)
        def _():
            pltpu.sync_copy(acc.at[pl.ds(15 * _RPT, _RWLAST)],
                            out_hbm.at[pl.ds(goff + 15 * _RPT, _RWLAST)])


@functools.cache
def _make_agg(interpret: bool = False):
    return pl.kernel(
        _agg_body,
        out_type=jax.ShapeDtypeStruct((_NG * _N, _W), jnp.float32),
        mesh=plsc.VectorSubcoreMesh(core_axis_name="c", subcore_axis_name="s"),
        scratch_types=[
            pltpu.VMEM((_NCH, _CH), jnp.int32),
            pltpu.VMEM((_NCH, _CH), jnp.int32),
            pltpu.VMEM((_CH,), jnp.int32),
            pltpu.VMEM((_CH,), jnp.int32),
            pltpu.VMEM((_CH,), jnp.int32),
            pltpu.VMEM((_CH,), jnp.int32),
            pltpu.VMEM((_CH,), jnp.int32),
            pltpu.VMEM((_CH,), jnp.int32),
            pltpu.VMEM((_CH, _W), jnp.float32),
            pltpu.VMEM((_CH, _W), jnp.float32),
            pltpu.VMEM((_CH, _W), jnp.float32),
            pltpu.VMEM_SHARED((_NA, _W), jnp.float32),
            pltpu.SemaphoreType.DMA,
            pltpu.SemaphoreType.DMA,
            pltpu.SemaphoreType.DMA,
            pltpu.SemaphoreType.DMA,
            pltpu.SemaphoreType.DMA,
            pltpu.SemaphoreType.DMA,
        ],
        compiler_params=pltpu.CompilerParams(use_tc_tiling_on_sc=False),
        interpret=interpret,
    )


def _cat(ref):
    return jnp.concatenate([ref[g] for g in range(_NG)], axis=1)


def _split(ref, h):
    for g in range(_NG):
        ref[g] = h[:, g * _W:(g + 1) * _W]


def _s1_body(s_ref, x_ref, c_ref, wl_ref, wr_ref, b_ref, o_ref):
    ic = 1.0 / jnp.maximum(c_ref[...], 1.0)
    S = _cat(s_ref)
    X = _cat(x_ref)
    h = (jnp.dot(S * ic, wl_ref[...], preferred_element_type=jnp.float32)
         + jnp.dot(X, wr_ref[...], preferred_element_type=jnp.float32)
         + b_ref[...])
    _split(o_ref, jnp.maximum(h, 0.0))


def _s2_body(s_ref, x_ref, c_ref, wl_ref, wr_ref, b_ref, u_ref, h_ref):
    cnt = c_ref[...]
    ic = 1.0 / jnp.maximum(cnt, 1.0)
    dinv = lax.rsqrt(cnt + 1.0)
    S = _cat(s_ref)
    X = _cat(x_ref)
    h = (jnp.dot(S * ic, wl_ref[...], preferred_element_type=jnp.float32)
         + jnp.dot(X, wr_ref[...], preferred_element_type=jnp.float32)
         + b_ref[...])
    _split(u_ref, dinv * h)
    _split(h_ref, h)


def _s3_body(g_ref, x_ref, c_ref, ws_ref, b_ref, v_ref, o_ref):
    dinv = lax.rsqrt(c_ref[...] + 1.0)
    Gm = _cat(g_ref)
    X = _cat(x_ref)
    P = dinv * Gm + (dinv * dinv) * X
    xs = jnp.maximum(
        jnp.dot(P, ws_ref[...], preferred_element_type=jnp.float32) + b_ref[...],
        0.0)
    _split(v_ref, dinv * xs)
    _split(o_ref, xs)


def _s4_body(g_ref, x_ref, c_ref, wm_ref, bm_ref, wv_ref, bv_ref, mu_ref, lv_ref):
    dinv = lax.rsqrt(c_ref[...] + 1.0)
    Gm = _cat(g_ref)
    X = _cat(x_ref)
    t = dinv * Gm + (dinv * dinv) * X
    mu_ref[...] = jnp.dot(t, wm_ref[...], preferred_element_type=jnp.float32) + bm_ref[...]
    lv_ref[...] = jnp.dot(t, wv_ref[...], preferred_element_type=jnp.float32) + bv_ref[...]


_FEAT = pl.BlockSpec((_NG, _RB, _W), lambda i: (0, i, 0))
_COL = pl.BlockSpec((_RB, 1), lambda i: (i, 0))
_WSPEC = pl.BlockSpec((_D, _D), lambda i: (0, 0))
_B = pl.BlockSpec((1, _D), lambda i: (0, 0))
_FULL = pl.BlockSpec((_RB, _D), lambda i: (i, 0))
_FSHAPE = jax.ShapeDtypeStruct((_NG, _N, _W), jnp.float32)
_OSHAPE = jax.ShapeDtypeStruct((_N, _D), jnp.float32)


@functools.cache
def _make_stages(interpret: bool = False):
    s1 = pl.pallas_call(
        _s1_body, grid=(_G,),
        in_specs=[_FEAT, _FEAT, _COL, _WSPEC, _WSPEC, _B],
        out_specs=_FEAT, out_shape=_FSHAPE, interpret=interpret)
    s2 = pl.pallas_call(
        _s2_body, grid=(_G,),
        in_specs=[_FEAT, _FEAT, _COL, _WSPEC, _WSPEC, _B],
        out_specs=(_FEAT, _FEAT), out_shape=(_FSHAPE, _FSHAPE),
        interpret=interpret)
    s3 = pl.pallas_call(
        _s3_body, grid=(_G,),
        in_specs=[_FEAT, _FEAT, _COL, _WSPEC, _B],
        out_specs=(_FEAT, _FEAT), out_shape=(_FSHAPE, _FSHAPE),
        interpret=interpret)
    s4 = pl.pallas_call(
        _s4_body, grid=(_G,),
        in_specs=[_FEAT, _FEAT, _COL, _WSPEC, _B, _WSPEC, _B],
        out_specs=(_FULL, _FULL), out_shape=(_OSHAPE, _OSHAPE),
        interpret=interpret)
    return s1, s2, s3, s4


def _run(x, edge_index, Wl1, bl1, Wr1, Wl2, bl2, Wr2, Ws, bs, Wm, bm, Wv, bv,
         interpret: bool = False):
    agg = _make_agg(interpret)
    s1, s2, s3, s4 = _make_stages(interpret)

    src = edge_index[0]
    dst = edge_index[1]
    srcp = jnp.concatenate(
        [src.reshape(_NT, _EPT),
         jnp.zeros((_NT, _PAD), jnp.int32)], axis=1).reshape(_NT, _NCH, _CH)
    dstp = jnp.concatenate(
        [dst.reshape(_NT, _EPT),
         jnp.full((_NT, _PAD), _N, jnp.int32)], axis=1).reshape(_NT, _NCH, _CH)
    zblk = jnp.zeros((_RPT, _W), jnp.float32)
    ones = jnp.ones((_NG * _N, _W), jnp.float32)

    X4 = jnp.moveaxis(x.reshape(_N, _NG, _W), 1, 0)

    cnt = agg(ones, srcp, dstp, zblk)[:_N, :1]
    S1 = agg(X4.reshape(_NG * _N, _W), srcp, dstp, zblk).reshape(_NG, _N, _W)
    H1 = s1(S1, X4, cnt, Wl1.T, Wr1.T, bl1.reshape(1, _D))
    S2 = agg(H1.reshape(_NG * _N, _W), srcp, dstp, zblk).reshape(_NG, _N, _W)
    U, H2 = s2(S2, H1, cnt, Wl2.T, Wr2.T, bl2.reshape(1, _D))
    Gm = agg(U.reshape(_NG * _N, _W), srcp, dstp, zblk).reshape(_NG, _N, _W)
    V, XS = s3(Gm, H2, cnt, Ws.T, bs.reshape(1, _D))
    Hh = agg(V.reshape(_NG * _N, _W), srcp, dstp, zblk).reshape(_NG, _N, _W)
    mu, lv = s4(Hh, XS, cnt, Wm.T, bm.reshape(1, _D), Wv.T, bv.reshape(1, _D))
    return (mu, lv)


def kernel(edge_emb_eq1, edge_index, Wl1, bl1, Wr1, Wl2, bl2, Wr2, Ws, bs,
           Wm, bm, Wv, bv):
    return _run(edge_emb_eq1, edge_index, Wl1, bl1, Wr1, Wl2, bl2, Wr2,
                Ws, bs, Wm, bm, Wv, bv)


# speedup vs baseline: 1.2714x; 1.1617x over previous
import functools

import jax
import jax.numpy as jnp
from jax import lax
from jax.experimental import pallas as pl
from jax.experimental.pallas import tpu as pltpu
from jax.experimental.pallas import tpu_sc as plsc

_N = 10000
_E = 160000
_D = 256
_W = 64
_NG = 4
_NT = 16
_NC = 2
_CH = 128
_EPT = _E // _NT
_NCH = 79
_PAD = _NCH * _CH - _EPT
_NA = _N + 8
_RPT = 632
_RZLAST = _NA - 15 * _RPT
_RWLAST = _N - 15 * _RPT
_RB = 1000
_G = _N // _RB


def _agg_body(x_hbm, src_hbm, dst_hbm, z_hbm, out_hbm,
              sidx, didx, s128a, d128a, s128b, d128b, s128c, d128c,
              rows_a, rows_b, rows_c, acc,
              gsem_a, gsem_b, gsem_c, ssem_a, ssem_b, ssem_c):
    c = lax.axis_index("c")
    s = lax.axis_index("s")
    pltpu.sync_copy(src_hbm.at[s], sidx)
    pltpu.sync_copy(dst_hbm.at[s], didx)

    bufs = ((s128a, d128a, rows_a, gsem_a, ssem_a),
            (s128b, d128b, rows_b, gsem_b, ssem_b),
            (s128c, d128c, rows_c, gsem_c, ssem_c))

    for ph in range(2):
        g = 2 * c + ph

        @pl.when(s < _NT - 1)
        def _():
            pltpu.sync_copy(z_hbm, acc.at[pl.ds(s * _RPT, _RPT)])

        @pl.when(s == _NT - 1)
        def _():
            pltpu.sync_copy(z_hbm.at[pl.ds(0, _RZLAST)],
                            acc.at[pl.ds(15 * _RPT, _RZLAST)])

        plsc.subcore_barrier()

        goff = g * _N

        def cidx(j, b):
            s128, d128 = b[0], b[1]
            for k in range(_CH // 16):
                s128[pl.ds(k * 16, 16)] = sidx[j, pl.ds(k * 16, 16)] + goff
                d128[pl.ds(k * 16, 16)] = didx[j, pl.ds(k * 16, 16)]

        def gat(j, b):
            cidx(j, b)
            pltpu.async_copy(x_hbm.at[b[0]], b[2], b[3])

        def wait_g(b):
            pltpu.make_async_copy(x_hbm.at[b[0]], b[2], b[3]).wait()

        def scat(b):
            pltpu.async_copy(b[2], acc.at[b[1]], b[4], add=True)

        def wait_s(b):
            pltpu.make_async_copy(b[2], acc.at[b[1]], b[4]).wait()

        gat(0, bufs[0])
        gat(1, bufs[1])
        wait_g(bufs[0])
        scat(bufs[0])
        gat(2, bufs[2])
        wait_g(bufs[1])
        scat(bufs[1])
        wait_s(bufs[0])
        gat(3, bufs[0])

        def tri(k, carry):
            j = 3 * k
            for t, (bj, bp) in enumerate(((2, 1), (0, 2), (1, 0))):
                jj = j + 2 + t
                wait_g(bufs[bj])
                scat(bufs[bj])
                wait_s(bufs[bp])
                gat(jj + 2, bufs[bp])
            return carry

        lax.fori_loop(0, (_NCH - 4) // 3, tri, 0)
        wait_g(bufs[2])
        scat(bufs[2])
        wait_g(bufs[0])
        scat(bufs[0])
        wait_s(bufs[1])
        wait_s(bufs[2])
        wait_s(bufs[0])
        plsc.subcore_barrier()

        @pl.when(s < _NT - 1)
        def _():
            pltpu.sync_copy(acc.at[pl.ds(s * _RPT, _RPT)],
                            out_hbm.at[pl.ds(goff + s * _RPT, _RPT)])

        @pl.when(s == _NT - 1)
        def _():
            pltpu.sync_copy(acc.at[pl.ds(15 * _RPT, _RWLAST)],
                            out_hbm.at[pl.ds(goff + 15 * _RPT, _RWLAST)])


@functools.cache
def _make_agg(interpret: bool = False):
    return pl.kernel(
        _agg_body,
        out_type=jax.ShapeDtypeStruct((_NG * _N, _W), jnp.float32),
        mesh=plsc.VectorSubcoreMesh(core_axis_name="c", subcore_axis_name="s"),
        scratch_types=[
            pltpu.VMEM((_NCH, _CH), jnp.int32),
            pltpu.VMEM((_NCH, _CH), jnp.int32),
            pltpu.VMEM((_CH,), jnp.int32),
            pltpu.VMEM((_CH,), jnp.int32),
            pltpu.VMEM((_CH,), jnp.int32),
            pltpu.VMEM((_CH,), jnp.int32),
            pltpu.VMEM((_CH,), jnp.int32),
            pltpu.VMEM((_CH,), jnp.int32),
            pltpu.VMEM((_CH, _W), jnp.float32),
            pltpu.VMEM((_CH, _W), jnp.float32),
            pltpu.VMEM((_CH, _W), jnp.float32),
            pltpu.VMEM_SHARED((_NA, _W), jnp.float32),
            pltpu.SemaphoreType.DMA,
            pltpu.SemaphoreType.DMA,
            pltpu.SemaphoreType.DMA,
            pltpu.SemaphoreType.DMA,
            pltpu.SemaphoreType.DMA,
            pltpu.SemaphoreType.DMA,
        ],
        compiler_params=pltpu.CompilerParams(use_tc_tiling_on_sc=False),
        interpret=interpret,
    )


_NCHD = 80


def _deg_body(dst_hbm, ones_hbm, z_hbm, out_hbm, didx, d128, ones_v, dacc):
    c = lax.axis_index("c")
    s = lax.axis_index("s")
    pltpu.sync_copy(dst_hbm.at[s], didx)
    pltpu.sync_copy(ones_hbm, ones_v)

    @pl.when(s < _NT - 1)
    def _():
        pltpu.sync_copy(z_hbm, dacc.at[pl.ds(s * _RPT, _RPT)])

    @pl.when(s == _NT - 1)
    def _():
        pltpu.sync_copy(z_hbm.at[pl.ds(0, _RZLAST)],
                        dacc.at[pl.ds(15 * _RPT, _RZLAST)])

    plsc.subcore_barrier()

    def step(i, carry):
        j = 2 * i + c
        for k in range(_CH // 16):
            d128[pl.ds(k * 16, 16)] = didx[j, pl.ds(k * 16, 16)]
        pltpu.sync_copy(ones_v, dacc.at[d128], add=True)
        return carry

    lax.fori_loop(0, _NCHD // 2, step, 0)
    plsc.subcore_barrier()

    @pl.when(s < _NT - 1)
    def _():
        pltpu.sync_copy(dacc.at[pl.ds(s * _RPT, _RPT)],
                        out_hbm.at[pl.ds(c * _N + s * _RPT, _RPT)])

    @pl.when(s == _NT - 1)
    def _():
        pltpu.sync_copy(dacc.at[pl.ds(15 * _RPT, _RWLAST)],
                        out_hbm.at[pl.ds(c * _N + 15 * _RPT, _RWLAST)])


@functools.cache
def _make_deg(interpret: bool = False):
    return pl.kernel(
        _deg_body,
        out_type=jax.ShapeDtypeStruct((_NC * _N, 16), jnp.float32),
        mesh=plsc.VectorSubcoreMesh(core_axis_name="c", subcore_axis_name="s"),
        scratch_types=[
            pltpu.VMEM((_NCHD, _CH), jnp.int32),
            pltpu.VMEM((_CH,), jnp.int32),
            pltpu.VMEM((_CH, 16), jnp.float32),
            pltpu.VMEM_SHARED((_NA, 16), jnp.float32),
        ],
        compiler_params=pltpu.CompilerParams(use_tc_tiling_on_sc=False),
        interpret=interpret,
    )


def _cat(ref):
    return jnp.concatenate([ref[g] for g in range(_NG)], axis=1)


def _split(ref, h):
    for g in range(_NG):
        ref[g] = h[:, g * _W:(g + 1) * _W]


def _s1_body(s_ref, x_ref, c_ref, wl_ref, wr_ref, b_ref, o_ref):
    ic = 1.0 / jnp.maximum(c_ref[0] + c_ref[1], 1.0)
    S = _cat(s_ref)
    X = _cat(x_ref)
    h = (jnp.dot(S * ic, wl_ref[...], preferred_element_type=jnp.float32)
         + jnp.dot(X, wr_ref[...], preferred_element_type=jnp.float32)
         + b_ref[...])
    _split(o_ref, jnp.maximum(h, 0.0))


def _s2_body(s_ref, x_ref, c_ref, wl_ref, wr_ref, b_ref, u_ref, h_ref):
    cnt = c_ref[0] + c_ref[1]
    ic = 1.0 / jnp.maximum(cnt, 1.0)
    dinv = lax.rsqrt(cnt + 1.0)
    S = _cat(s_ref)
    X = _cat(x_ref)
    h = (jnp.dot(S * ic, wl_ref[...], preferred_element_type=jnp.float32)
         + jnp.dot(X, wr_ref[...], preferred_element_type=jnp.float32)
         + b_ref[...])
    _split(u_ref, dinv * h)
    _split(h_ref, h)


def _s3_body(g_ref, x_ref, c_ref, ws_ref, b_ref, v_ref, o_ref):
    dinv = lax.rsqrt(c_ref[0] + c_ref[1] + 1.0)
    Gm = _cat(g_ref)
    X = _cat(x_ref)
    P = dinv * Gm + (dinv * dinv) * X
    xs = jnp.maximum(
        jnp.dot(P, ws_ref[...], preferred_element_type=jnp.float32) + b_ref[...],
        0.0)
    _split(v_ref, dinv * xs)
    _split(o_ref, xs)


def _s4_body(g_ref, x_ref, c_ref, wm_ref, bm_ref, wv_ref, bv_ref, mu_ref, lv_ref):
    dinv = lax.rsqrt(c_ref[0] + c_ref[1] + 1.0)
    Gm = _cat(g_ref)
    X = _cat(x_ref)
    t = dinv * Gm + (dinv * dinv) * X
    mu_ref[...] = jnp.dot(t, wm_ref[...], preferred_element_type=jnp.float32) + bm_ref[...]
    lv_ref[...] = jnp.dot(t, wv_ref[...], preferred_element_type=jnp.float32) + bv_ref[...]


_FEAT = pl.BlockSpec((_NG, _RB, _W), lambda i: (0, i, 0))
_COL = pl.BlockSpec((_NC, _RB, 1), lambda i: (0, i, 0))
_WSPEC = pl.BlockSpec((_D, _D), lambda i: (0, 0))
_B = pl.BlockSpec((1, _D), lambda i: (0, 0))
_FULL = pl.BlockSpec((_RB, _D), lambda i: (i, 0))
_FSHAPE = jax.ShapeDtypeStruct((_NG, _N, _W), jnp.float32)
_OSHAPE = jax.ShapeDtypeStruct((_N, _D), jnp.float32)


@functools.cache
def _make_stages(interpret: bool = False):
    s1 = pl.pallas_call(
        _s1_body, grid=(_G,),
        in_specs=[_FEAT, _FEAT, _COL, _WSPEC, _WSPEC, _B],
        out_specs=_FEAT, out_shape=_FSHAPE, interpret=interpret)
    s2 = pl.pallas_call(
        _s2_body, grid=(_G,),
        in_specs=[_FEAT, _FEAT, _COL, _WSPEC, _WSPEC, _B],
        out_specs=(_FEAT, _FEAT), out_shape=(_FSHAPE, _FSHAPE),
        interpret=interpret)
    s3 = pl.pallas_call(
        _s3_body, grid=(_G,),
        in_specs=[_FEAT, _FEAT, _COL, _WSPEC, _B],
        out_specs=(_FEAT, _FEAT), out_shape=(_FSHAPE, _FSHAPE),
        interpret=interpret)
    s4 = pl.pallas_call(
        _s4_body, grid=(_G,),
        in_specs=[_FEAT, _FEAT, _COL, _WSPEC, _B, _WSPEC, _B],
        out_specs=(_FULL, _FULL), out_shape=(_OSHAPE, _OSHAPE),
        interpret=interpret)
    return s1, s2, s3, s4


def _run(x, edge_index, Wl1, bl1, Wr1, Wl2, bl2, Wr2, Ws, bs, Wm, bm, Wv, bv,
         interpret: bool = False):
    agg = _make_agg(interpret)
    deg = _make_deg(interpret)
    s1, s2, s3, s4 = _make_stages(interpret)

    src = edge_index[0]
    dst = edge_index[1]
    srcp = jnp.concatenate(
        [src.reshape(_NT, _EPT),
         jnp.zeros((_NT, _PAD), jnp.int32)], axis=1).reshape(_NT, _NCH, _CH)
    dstp = jnp.concatenate(
        [dst.reshape(_NT, _EPT),
         jnp.full((_NT, _PAD), _N, jnp.int32)], axis=1).reshape(_NT, _NCH, _CH)
    dstp_deg = jnp.concatenate(
        [dst.reshape(_NT, _EPT),
         jnp.full((_NT, _NCHD * _CH - _EPT), _N, jnp.int32)],
        axis=1).reshape(_NT, _NCHD, _CH)
    zblk = jnp.zeros((_RPT, _W), jnp.float32)
    z16 = jnp.zeros((_RPT, 16), jnp.float32)
    ones16 = jnp.ones((_CH, 16), jnp.float32)

    X4 = jnp.moveaxis(x.reshape(_N, _NG, _W), 1, 0)

    cnt = deg(dstp_deg, ones16, z16).reshape(_NC, _N, 16)[:, :, :1]
    S1 = agg(X4.reshape(_NG * _N, _W), srcp, dstp, zblk).reshape(_NG, _N, _W)
    H1 = s1(S1, X4, cnt, Wl1.T, Wr1.T, bl1.reshape(1, _D))
    S2 = agg(H1.reshape(_NG * _N, _W), srcp, dstp, zblk).reshape(_NG, _N, _W)
    U, H2 = s2(S2, H1, cnt, Wl2.T, Wr2.T, bl2.reshape(1, _D))
    Gm = agg(U.reshape(_NG * _N, _W), srcp, dstp, zblk).reshape(_NG, _N, _W)
    V, XS = s3(Gm, H2, cnt, Ws.T, bs.reshape(1, _D))
    Hh = agg(V.reshape(_NG * _N, _W), srcp, dstp, zblk).reshape(_NG, _N, _W)
    mu, lv = s4(Hh, XS, cnt, Wm.T, bm.reshape(1, _D), Wv.T, bv.reshape(1, _D))
    return (mu, lv)


def kernel(edge_emb_eq1, edge_index, Wl1, bl1, Wr1, Wl2, bl2, Wr2, Ws, bs,
           Wm, bm, Wv, bv):
    return _run(edge_emb_eq1, edge_index, Wl1, bl1, Wr1, Wl2, bl2, Wr2,
                Ws, bs, Wm, bm, Wv, bv)
